# baseline (speedup 1.0000x reference)
import jax
import jax.numpy as jnp
from jax.experimental import pallas as pl
from jax.experimental.pallas import tpu as pltpu

_N = 4096
_CE = 4
_CN = 128
_COUT = 128
_LANES = 128
_SUB = (_N * _CE) // _LANES

_BI = 128
_NI = _N // _BI


def _body(a_ref, w2_ref, x_ref, wself_ref, b_ref, dinv_ref, o_ref):
    acc = jnp.sum(a_ref[...], axis=1)
    msg = (
        jnp.dot(acc, w2_ref[...], preferred_element_type=jnp.float32)
        * dinv_ref[...]
    )
    self_t = jnp.dot(
        x_ref[...], wself_ref[...], preferred_element_type=jnp.float32
    )
    o_ref[...] = jnp.maximum(msg + self_t + b_ref[...], 0.0)


def kernel(D, A, X, W_pass, b_pass, W_self, b_self):
    A3 = A.reshape(_N, _SUB, _LANES)
    W2 = jnp.tile(W_pass.T, (_LANES // _CE, 1))
    Wself_T = W_self.T
    b = (b_pass + b_self).reshape(1, _COUT)
    Dinv = (1.0 / D).reshape(_N, 1)

    out = pl.pallas_call(
        _body,
        grid=(_NI,),
        in_specs=[
            pl.BlockSpec((_BI, _SUB, _LANES), lambda i: (i, 0, 0)),
            pl.BlockSpec((_LANES, _COUT), lambda i: (0, 0)),
            pl.BlockSpec((_BI, _CN), lambda i: (i, 0)),
            pl.BlockSpec((_CN, _COUT), lambda i: (0, 0)),
            pl.BlockSpec((1, _COUT), lambda i: (0, 0)),
            pl.BlockSpec((_BI, 1), lambda i: (i, 0)),
        ],
        out_specs=pl.BlockSpec((_BI, _COUT), lambda i: (i, 0)),
        out_shape=jax.ShapeDtypeStruct((_N, _COUT), jnp.float32),
        compiler_params=pltpu.CompilerParams(
            dimension_semantics=("arbitrary",),
        ),
    )(A3, W2, X, Wself_T, b, Dinv)
    return out


# speedup vs baseline: 9.1815x; 9.1815x over previous
import jax
import jax.numpy as jnp
from jax.experimental import pallas as pl
from jax.experimental.pallas import tpu as pltpu

_N = 4096
_CE = 4
_CN = 128
_COUT = 128

_BI = 128
_NI = _N // _BI


def _body(a_ref, wp_ref, x_ref, wself_ref, b_ref, dinv_ref, o_ref):
    a = a_ref[...]
    w = _N
    while w > 128:
        w //= 2
        a = a[:, :, :w] + a[:, :, w:]
    acc = jnp.sum(a, axis=2)
    msg = (
        jnp.dot(acc, wp_ref[...], preferred_element_type=jnp.float32)
        * dinv_ref[...]
    )
    self_t = jnp.dot(
        x_ref[...], wself_ref[...], preferred_element_type=jnp.float32
    )
    o_ref[...] = jnp.maximum(msg + self_t + b_ref[...], 0.0)


def kernel(D, A, X, W_pass, b_pass, W_self, b_self):
    At = jnp.transpose(A, (0, 2, 1))
    Wp_T = W_pass.T
    Wself_T = W_self.T
    b = (b_pass + b_self).reshape(1, _COUT)
    Dinv = (1.0 / D).reshape(_N, 1)

    out = pl.pallas_call(
        _body,
        grid=(_NI,),
        in_specs=[
            pl.BlockSpec((_BI, _CE, _N), lambda i: (i, 0, 0)),
            pl.BlockSpec((_CE, _COUT), lambda i: (0, 0)),
            pl.BlockSpec((_BI, _CN), lambda i: (i, 0)),
            pl.BlockSpec((_CN, _COUT), lambda i: (0, 0)),
            pl.BlockSpec((1, _COUT), lambda i: (0, 0)),
            pl.BlockSpec((_BI, 1), lambda i: (i, 0)),
        ],
        out_specs=pl.BlockSpec((_BI, _COUT), lambda i: (i, 0)),
        out_shape=jax.ShapeDtypeStruct((_N, _COUT), jnp.float32),
        compiler_params=pltpu.CompilerParams(
            dimension_semantics=("arbitrary",),
        ),
    )(At, Wp_T, X, Wself_T, b, Dinv)
    return out
